# baseline (speedup 1.0000x reference)
import functools

import jax
import jax.numpy as jnp
import numpy as np
from jax import lax
from jax.experimental import pallas as pl
from jax.experimental.pallas import tpu as pltpu
from jax.experimental.pallas import tpu_sc as plsc

_ALPHA = 0.5
_LAM = float(np.random.RandomState(0).beta(_ALPHA, 1.0 - _ALPHA))

_B = 256
_RT = 128
_SL = 672
_LN = 224
_CS = 56
_NC = _SL // _CS
_NW = 32
_RPW = (_B - _RT) // _NW
_NT = _RPW * _NC


def _tc_body(idx_ref, x_ref, xp_ref, o_ref):
    o_ref[...] = _LAM * x_ref[...] + (1.0 - _LAM) * xp_ref[...]


def _sc_body(x2, y128, idx, out, yb128,
             idxv, a0, p0, o0, a1, p1, o1, iv, ybuf,
             sa0, sp0, so0, sa1, sp1, so1):
    wid = lax.axis_index("s") * 2 + lax.axis_index("c")
    rbase = _RT + wid * _RPW
    roff = (wid % 2) * _RPW
    obase = wid * _RPW * _SL

    awin = _RT + (wid // 2) * 8
    pltpu.sync_copy(idx.at[pl.ds(awin, 16)], idxv)
    vidx = idxv[...]
    svals = [vidx[k] for k in range(8)]

    def _ridx(r):
        rr = r + roff
        acc = svals[0]
        for k in range(1, 8):
            acc = jnp.where(rr == k, svals[k], acc)
        return acc

    @pl.when(wid == 0)
    def _():
        pltpu.sync_copy(idx.at[pl.ds(0, _B)], iv)
        for h in range(2):
            pltpu.async_copy(y128.at[iv.at[pl.ds(h * 128, 128)]], ybuf, sa0).wait()
            pltpu.sync_copy(ybuf, yb128.at[pl.ds(h * 128, 128)])

    slots = ((a0, p0, o0, sa0, sp0, so0), (a1, p1, o1, sa1, sp1, so1))

    def in_copies(t, slot):
        a, p, _, sa, sp, _ = slot
        r = t // _NC
        c = t % _NC
        ridx = _ridx(r)
        lin = pltpu.make_async_copy(
            x2.at[pl.ds((rbase + r) * _SL + c * _CS, _CS)], a, sa)
        gat = pltpu.make_async_copy(
            x2.at[pl.ds(ridx * _SL + c * _CS, _CS)], p, sp)
        return lin, gat

    def st_copy(t, slot):
        _, _, o, _, _, so = slot
        return pltpu.make_async_copy(
            o, out.at[pl.ds(obase + (t // _NC) * _SL + (t % _NC) * _CS, _CS)], so)

    for s in range(2):
        lin, gat = in_copies(s, slots[s])
        lin.start()
        gat.start()

    @pl.loop(0, _NT, step=2)
    def _tiles(i):
        for s in range(2):
            t = i + s
            a, p, o, _, _, _ = slots[s]
            lin, gat = in_copies(t, slots[s])
            lin.wait()
            gat.wait()

            @pl.when(i >= 2)
            def _():
                st_copy(t - 2, slots[s]).wait()

            @pl.loop(0, _CS)
            def _row(q):
                for j in range(_LN // 16):
                    av = a[q, pl.ds(j * 16, 16)]
                    pv = p[q, pl.ds(j * 16, 16)]
                    o[q, pl.ds(j * 16, 16)] = av * _LAM + pv * (1.0 - _LAM)

            st_copy(t, slots[s]).start()

            @pl.when(i < _NT - 2)
            def _():
                lin2, gat2 = in_copies(t + 2, slots[s])
                lin2.start()
                gat2.start()

    for s in range(2):
        st_copy(_NT - 2 + s, slots[s]).wait()


def kernel(x, y, index):
    S = 1176
    x3 = x.reshape(_B, S, 128)
    tc_out = pl.pallas_call(
        _tc_body,
        grid_spec=pltpu.PrefetchScalarGridSpec(
            num_scalar_prefetch=1,
            grid=(_RT,),
            in_specs=[
                pl.BlockSpec((1, S, 128), lambda i, idx: (i, 0, 0)),
                pl.BlockSpec((1, S, 128), lambda i, idx: (idx[i], 0, 0)),
            ],
            out_specs=pl.BlockSpec((1, S, 128), lambda i, idx: (i, 0, 0)),
        ),
        out_shape=jax.ShapeDtypeStruct((_RT, S, 128), jnp.float32),
    )(index, x3, x3)

    x2 = x.reshape(_B * _SL, _LN)
    y128 = jnp.broadcast_to(y[:, None], (_B, 128))
    idxp = jnp.concatenate([index.astype(jnp.int32), jnp.zeros((8,), jnp.int32)])

    mesh = plsc.VectorSubcoreMesh(core_axis_name="c", subcore_axis_name="s")
    sc_out, yb128 = pl.kernel(
        _sc_body,
        out_type=(
            jax.ShapeDtypeStruct(((_B - _RT) * _SL, _LN), jnp.float32),
            jax.ShapeDtypeStruct((_B, 128), jnp.int32),
        ),
        mesh=mesh,
        scratch_types=[
            pltpu.VMEM((16,), jnp.int32),
            pltpu.VMEM((_CS, _LN), jnp.float32),
            pltpu.VMEM((_CS, _LN), jnp.float32),
            pltpu.VMEM((_CS, _LN), jnp.float32),
            pltpu.VMEM((_CS, _LN), jnp.float32),
            pltpu.VMEM((_CS, _LN), jnp.float32),
            pltpu.VMEM((_CS, _LN), jnp.float32),
            pltpu.VMEM((_B,), jnp.int32),
            pltpu.VMEM((128, 128), jnp.int32),
            pltpu.SemaphoreType.DMA,
            pltpu.SemaphoreType.DMA,
            pltpu.SemaphoreType.DMA,
            pltpu.SemaphoreType.DMA,
            pltpu.SemaphoreType.DMA,
            pltpu.SemaphoreType.DMA,
        ],
    )(x2, y128, idxp)

    mixed = jnp.concatenate(
        [tc_out.reshape(_RT, 3, 224, 224),
         sc_out.reshape(_B - _RT, 3, 224, 224)], axis=0)
    yb = yb128[:, 0]
    return (mixed, y, yb, jnp.float32(_LAM))


# speedup vs baseline: 1.5934x; 1.5934x over previous
import jax
import jax.numpy as jnp
import numpy as np
from jax.experimental import pallas as pl
from jax.experimental.pallas import tpu as pltpu

_ALPHA = 0.5
_LAM = float(np.random.RandomState(0).beta(_ALPHA, 1.0 - _ALPHA))


def _mix_body(idx_ref, x_ref, g0_ref, g1_ref, o_ref):
    o_ref[0] = _LAM * x_ref[0] + (1.0 - _LAM) * g0_ref[0]
    o_ref[1] = _LAM * x_ref[1] + (1.0 - _LAM) * g1_ref[0]


def kernel(x, y, index):
    B = x.shape[0]
    S = 1176
    x3 = x.reshape(B, S, 128)
    out = pl.pallas_call(
        _mix_body,
        grid_spec=pltpu.PrefetchScalarGridSpec(
            num_scalar_prefetch=1,
            grid=(B // 2,),
            in_specs=[
                pl.BlockSpec((2, S, 128), lambda i, idx: (i, 0, 0)),
                pl.BlockSpec((1, S, 128), lambda i, idx: (idx[2 * i], 0, 0)),
                pl.BlockSpec((1, S, 128), lambda i, idx: (idx[2 * i + 1], 0, 0)),
            ],
            out_specs=pl.BlockSpec((2, S, 128), lambda i, idx: (i, 0, 0)),
        ),
        out_shape=jax.ShapeDtypeStruct((B, S, 128), jnp.float32),
    )(index, x3, x3, x3)
    mixed = out.reshape(x.shape)
    y_b = jnp.take(y, index, axis=0)
    return (mixed, y, y_b, jnp.float32(_LAM))


# speedup vs baseline: 1.6735x; 1.0503x over previous
import jax
import jax.numpy as jnp
import numpy as np
from jax.experimental import pallas as pl
from jax.experimental.pallas import tpu as pltpu

_ALPHA = 0.5
_LAM = float(np.random.RandomState(0).beta(_ALPHA, 1.0 - _ALPHA))

_R = 4


def _mix_body(idx_ref, x_ref, *refs):
    g_refs = refs[:_R]
    o_ref = refs[_R]
    for r in range(_R):
        o_ref[r] = _LAM * x_ref[r] + (1.0 - _LAM) * g_refs[r][0]


def kernel(x, y, index):
    B = x.shape[0]
    S = 1176
    x3 = x.reshape(B, S, 128)

    def _gspec(r):
        return pl.BlockSpec((1, S, 128), lambda i, idx, r=r: (idx[_R * i + r], 0, 0))

    out = pl.pallas_call(
        _mix_body,
        grid_spec=pltpu.PrefetchScalarGridSpec(
            num_scalar_prefetch=1,
            grid=(B // _R,),
            in_specs=[pl.BlockSpec((_R, S, 128), lambda i, idx: (i, 0, 0))]
            + [_gspec(r) for r in range(_R)],
            out_specs=pl.BlockSpec((_R, S, 128), lambda i, idx: (i, 0, 0)),
        ),
        out_shape=jax.ShapeDtypeStruct((B, S, 128), jnp.float32),
    )(index, x3, *([x3] * _R))
    mixed = out.reshape(x.shape)
    y_b = jnp.take(y, index, axis=0)
    return (mixed, y, y_b, jnp.float32(_LAM))


# speedup vs baseline: 1.6819x; 1.0050x over previous
import jax
import jax.numpy as jnp
import numpy as np
from jax.experimental import pallas as pl
from jax.experimental.pallas import tpu as pltpu

_ALPHA = 0.5
_LAM = float(np.random.RandomState(0).beta(_ALPHA, 1.0 - _ALPHA))

_R = 8


def _mix_body(idx_ref, x_ref, *refs):
    g_refs = refs[:_R]
    o_ref = refs[_R]
    for r in range(_R):
        o_ref[r] = _LAM * x_ref[r] + (1.0 - _LAM) * g_refs[r][0]


def kernel(x, y, index):
    B = x.shape[0]
    S = 1176
    x3 = x.reshape(B, S, 128)

    def _gspec(r):
        return pl.BlockSpec((1, S, 128), lambda i, idx, r=r: (idx[_R * i + r], 0, 0))

    out = pl.pallas_call(
        _mix_body,
        grid_spec=pltpu.PrefetchScalarGridSpec(
            num_scalar_prefetch=1,
            grid=(B // _R,),
            in_specs=[pl.BlockSpec((_R, S, 128), lambda i, idx: (i, 0, 0))]
            + [_gspec(r) for r in range(_R)],
            out_specs=pl.BlockSpec((_R, S, 128), lambda i, idx: (i, 0, 0)),
        ),
        out_shape=jax.ShapeDtypeStruct((B, S, 128), jnp.float32),
    )(index, x3, *([x3] * _R))
    mixed = out.reshape(x.shape)
    y_b = jnp.take(y, index, axis=0)
    return (mixed, y, y_b, jnp.float32(_LAM))
